# baseline (speedup 1.0000x reference)
import functools

import jax
import jax.numpy as jnp
from jax import lax
from jax.experimental import pallas as pl
from jax.experimental.pallas import tpu as pltpu
from jax.experimental.pallas import tpu_sc as plsc

N, D, E = 10000, 128, 320000
DH = D // 2
NC, NS = 2, 16
NW = NC * NS
B = 128
CPT = 170
EP = NS * B * CPT
CPTD = EP // (NW * B)
NBUF = 5
NP = 10240
RPS = NP // NS
DEGW = 16

_mesh = plsc.VectorSubcoreMesh(
    core_axis_name="c", subcore_axis_name="s", num_cores=NC, num_subcores=NS)
_sc_params = pltpu.CompilerParams(use_tc_tiling_on_sc=False)


@functools.partial(
    pl.kernel,
    out_type=jax.ShapeDtypeStruct((NC, NP, DEGW), jnp.float32),
    mesh=_mesh,
    scratch_types=[
        pltpu.VMEM((B, DEGW), jnp.float32),
        pltpu.VMEM((CPTD, B), jnp.int32),
        pltpu.VMEM_SHARED((NP, DEGW), jnp.float32),
    ],
    compiler_params=_sc_params,
)
def _deg_sc(dst_hbm, ones_hbm, zeros_hbm, out_hbm, ones_v, dst_v, acc):
    c = lax.axis_index("c")
    s = lax.axis_index("s")
    wid = c * NS + s
    pltpu.sync_copy(zeros_hbm.at[pl.ds(s * RPS, RPS)], acc.at[pl.ds(s * RPS, RPS)])
    pltpu.sync_copy(ones_hbm, ones_v)
    pltpu.sync_copy(dst_hbm.at[pl.ds(wid * CPTD, CPTD)], dst_v)
    plsc.subcore_barrier()

    def body(j, carry):
        pltpu.sync_copy(ones_v, acc.at[dst_v.at[j]], add=True)
        return carry

    lax.fori_loop(0, CPTD, body, 0)
    plsc.subcore_barrier()
    pltpu.sync_copy(acc.at[pl.ds(s * RPS, RPS)],
                    out_hbm.at[c, pl.ds(s * RPS, RPS)])


@functools.partial(
    pl.kernel,
    out_type=jax.ShapeDtypeStruct((NC, NP, DH), jnp.float32),
    mesh=_mesh,
    scratch_types=(
        [pltpu.VMEM((CPT, B), jnp.int32),
         pltpu.VMEM((CPT, B), jnp.int32)]
        + [pltpu.VMEM((B, DH), jnp.float32) for _ in range(NBUF)]
        + [pltpu.VMEM_SHARED((NP, DH), jnp.float32)]
        + [pltpu.SemaphoreType.DMA for _ in range(2 * NBUF)]
    ),
    compiler_params=_sc_params,
)
def _agg_sc(g_hbm, src_hbm, dst_hbm, zeros_hbm, out_hbm,
            src_v, dst_v, *rest):
    bufs = rest[:NBUF]
    acc = rest[NBUF]
    gsem = rest[NBUF + 1:NBUF + 1 + NBUF]
    ssem = rest[NBUF + 1 + NBUF:]
    c = lax.axis_index("c")
    s = lax.axis_index("s")
    gh = g_hbm.at[c]
    pltpu.sync_copy(zeros_hbm.at[pl.ds(s * RPS, RPS)], acc.at[pl.ds(s * RPS, RPS)])
    pltpu.sync_copy(src_hbm.at[pl.ds(s * CPT, CPT)], src_v)
    pltpu.sync_copy(dst_hbm.at[pl.ds(s * CPT, CPT)], dst_v)
    plsc.subcore_barrier()

    def issue_gather(j, b):
        pltpu.async_copy(gh.at[src_v.at[j]], bufs[b], gsem[b])

    def wait_gather(j, b):
        pltpu.make_async_copy(gh.at[src_v.at[j]], bufs[b], gsem[b]).wait()

    def issue_scatter(j, b):
        pltpu.async_copy(bufs[b], acc.at[dst_v.at[j]], ssem[b], add=True)

    def wait_scatter(j, b):
        pltpu.make_async_copy(bufs[b], acc.at[dst_v.at[j]], ssem[b]).wait()

    def steps(j, b):
        wait_gather(j, b)
        issue_scatter(j, b)
        wait_scatter(j - 2, (b - 2) % NBUF)
        issue_gather(j + 3, (b + 3) % NBUF)

    for j in range(3):
        issue_gather(j, j)
    for j in range(2):
        wait_gather(j, j)
        issue_scatter(j, j)
        issue_gather(j + 3, (j + 3) % NBUF)
    for j in range(2, NBUF):
        steps(j, j)

    def body(o, carry):
        j0 = o * NBUF
        for b in range(NBUF):
            steps(j0 + b, b)
        return carry

    lax.fori_loop(1, CPT // NBUF - 1, body, 0)

    for j in range(CPT - NBUF, CPT):
        b = j % NBUF
        if j + 3 < CPT:
            steps(j, b)
        else:
            wait_gather(j, b)
            issue_scatter(j, b)
            wait_scatter(j - 2, (b - 2) % NBUF)
    for j in range(CPT - 2, CPT):
        wait_scatter(j, j % NBUF)

    plsc.subcore_barrier()
    pltpu.sync_copy(acc.at[pl.ds(s * RPS, RPS)],
                    out_hbm.at[c, pl.ds(s * RPS, RPS)])


NB = 10
RB = N // NB

_blk = pl.BlockSpec((RB, D), lambda i: (i, 0))
_blkh = pl.BlockSpec((NC, RB, DH), lambda i: (0, i, 0))
_blkdeg = pl.BlockSpec((RB, DEGW), lambda i: (i, 0))
_blkw = pl.BlockSpec((D, D), lambda i: (0, 0))
_blkb = pl.BlockSpec((1, D), lambda i: (0, 0))
_tc_params = pltpu.CompilerParams(dimension_semantics=("parallel",))


def _dinv_of(dp0_ref, dp1_ref):
    return lax.rsqrt(1.0 + dp0_ref[:, 0:1] + dp1_ref[:, 0:1])


def _split(h):
    return jnp.stack([h[:, :DH], h[:, DH:]], axis=0)


def _tc_first_body(dp0_ref, dp1_ref, x_ref, w_ref, o_ref):
    dinv = _dinv_of(dp0_ref, dp1_ref)
    h = jnp.dot(x_ref[...], w_ref[...], preferred_element_type=jnp.float32)
    o_ref[...] = _split(h * dinv)


def _tc_mid_body(dp0_ref, dp1_ref, g_ref, s_ref, w_ref, b_ref, o_ref):
    dinv = _dinv_of(dp0_ref, dp1_ref)
    t = g_ref[...] + s_ref[...]
    z = dinv * jnp.concatenate([t[0], t[1]], axis=-1) + b_ref[...]
    z = jnp.maximum(z, 0.0)
    h = jnp.dot(z, w_ref[...], preferred_element_type=jnp.float32)
    o_ref[...] = _split(h * dinv)


def _tc_last_body(dp0_ref, dp1_ref, g_ref, s_ref, w_ref, b_ref, bh_ref, o_ref):
    dinv = _dinv_of(dp0_ref, dp1_ref)
    t = g_ref[...] + s_ref[...]
    z = dinv * jnp.concatenate([t[0], t[1]], axis=-1) + b_ref[...]
    h = jnp.dot(z, w_ref[...], preferred_element_type=jnp.float32)
    o_ref[...] = h + bh_ref[...]


_out_split = jax.ShapeDtypeStruct((NC, N, DH), jnp.float32)

_tc_first = pl.pallas_call(
    _tc_first_body,
    grid=(NB,),
    in_specs=[_blkdeg, _blkdeg, _blk, _blkw],
    out_specs=_blkh,
    out_shape=_out_split,
    compiler_params=_tc_params,
)

_tc_mid = pl.pallas_call(
    _tc_mid_body,
    grid=(NB,),
    in_specs=[_blkdeg, _blkdeg, _blkh, _blkh, _blkw, _blkb],
    out_specs=_blkh,
    out_shape=_out_split,
    compiler_params=_tc_params,
)

_tc_last = pl.pallas_call(
    _tc_last_body,
    grid=(NB,),
    in_specs=[_blkdeg, _blkdeg, _blkh, _blkh, _blkw, _blkb, _blkb],
    out_specs=_blk,
    out_shape=jax.ShapeDtypeStruct((N, D), jnp.float32),
    compiler_params=_tc_params,
)


def kernel(x, edge_index, W1, b1, W2, b2, W3, b3, Wh, bh):
    pad = EP - E
    src2 = jnp.concatenate(
        [edge_index[0], jnp.zeros((pad,), jnp.int32)]).reshape(EP // B, B)
    dst2 = jnp.concatenate(
        [edge_index[1], jnp.full((pad,), N, jnp.int32)]).reshape(EP // B, B)
    zeros_d = jnp.zeros((NP, DH), jnp.float32)
    zeros_w = jnp.zeros((NP, DEGW), jnp.float32)
    ones_w = jnp.ones((B, DEGW), jnp.float32)
    b1r = b1.reshape(1, D)
    b2r = b2.reshape(1, D)
    b3r = b3.reshape(1, D)
    bhr = bh.reshape(1, D)

    degp = _deg_sc(dst2, ones_w, zeros_w)
    dp0, dp1 = degp[0, :N], degp[1, :N]

    g1 = _tc_first(dp0, dp1, x, W1)
    s1 = _agg_sc(g1, src2, dst2, zeros_d)
    g2 = _tc_mid(dp0, dp1, g1, s1[:, :N], W2, b1r)
    s2 = _agg_sc(g2, src2, dst2, zeros_d)
    g3 = _tc_mid(dp0, dp1, g2, s2[:, :N], W3, b2r)
    s3 = _agg_sc(g3, src2, dst2, zeros_d)
    out = _tc_last(dp0, dp1, g3, s3[:, :N], Wh, b3r, bhr)
    return out


# speedup vs baseline: 3.1110x; 3.1110x over previous
import functools

import jax
import jax.numpy as jnp
from jax import lax
from jax.experimental import pallas as pl
from jax.experimental.pallas import tpu as pltpu
from jax.experimental.pallas import tpu_sc as plsc

N, D, E = 10000, 128, 320000
NC, NS = 2, 16
NW = NC * NS
B = 112
CPT = 90
EP = NW * B * CPT
NBUF = 2
NP = 10016
RPS = NP // NS
DEGW = 16

_mesh = plsc.VectorSubcoreMesh(
    core_axis_name="c", subcore_axis_name="s", num_cores=NC, num_subcores=NS)
_sc_params = pltpu.CompilerParams(use_tc_tiling_on_sc=False)


@functools.partial(
    pl.kernel,
    out_type=jax.ShapeDtypeStruct((NC, NP, DEGW), jnp.float32),
    mesh=_mesh,
    scratch_types=[
        pltpu.VMEM((B, DEGW), jnp.float32),
        pltpu.VMEM((CPT, B), jnp.int32),
        pltpu.VMEM_SHARED((NP, DEGW), jnp.float32),
    ],
    compiler_params=_sc_params,
)
def _deg_sc(dst_hbm, ones_hbm, zeros_hbm, out_hbm, ones_v, dst_v, acc):
    c = lax.axis_index("c")
    s = lax.axis_index("s")
    wid = c * NS + s
    pltpu.sync_copy(zeros_hbm.at[pl.ds(s * RPS, RPS)], acc.at[pl.ds(s * RPS, RPS)])
    pltpu.sync_copy(ones_hbm, ones_v)
    pltpu.sync_copy(dst_hbm.at[pl.ds(wid * CPT, CPT)], dst_v)
    plsc.subcore_barrier()

    def body(j, carry):
        pltpu.sync_copy(ones_v, acc.at[dst_v.at[j]], add=True)
        return carry

    lax.fori_loop(0, CPT, body, 0)
    plsc.subcore_barrier()
    pltpu.sync_copy(acc.at[pl.ds(s * RPS, RPS)],
                    out_hbm.at[c, pl.ds(s * RPS, RPS)])


@functools.partial(
    pl.kernel,
    out_type=jax.ShapeDtypeStruct((NC, NP, D), jnp.float32),
    mesh=_mesh,
    scratch_types=(
        [pltpu.VMEM((CPT, B), jnp.int32),
         pltpu.VMEM((CPT, B), jnp.int32)]
        + [pltpu.VMEM((B, D), jnp.float32) for _ in range(NBUF)]
        + [pltpu.VMEM_SHARED((NP, D), jnp.float32)]
        + [pltpu.SemaphoreType.DMA for _ in range(2 * NBUF)]
    ),
    compiler_params=_sc_params,
)
def _agg_sc(g_hbm, src_hbm, dst_hbm, zeros_hbm, out_hbm,
            src_v, dst_v, *rest):
    bufs = rest[:NBUF]
    acc = rest[NBUF]
    gsem = rest[NBUF + 1:NBUF + 1 + NBUF]
    ssem = rest[NBUF + 1 + NBUF:]
    c = lax.axis_index("c")
    s = lax.axis_index("s")
    wid = c * NS + s
    pltpu.sync_copy(zeros_hbm.at[pl.ds(s * RPS, RPS)], acc.at[pl.ds(s * RPS, RPS)])
    pltpu.sync_copy(src_hbm.at[pl.ds(wid * CPT, CPT)], src_v)
    pltpu.sync_copy(dst_hbm.at[pl.ds(wid * CPT, CPT)], dst_v)
    plsc.subcore_barrier()

    def issue_gather(j, b):
        pltpu.async_copy(g_hbm.at[src_v.at[j]], bufs[b], gsem[b])

    def wait_gather(j, b):
        pltpu.make_async_copy(g_hbm.at[src_v.at[j]], bufs[b], gsem[b]).wait()

    def issue_scatter(j, b):
        pltpu.async_copy(bufs[b], acc.at[dst_v.at[j]], ssem[b], add=True)

    def wait_scatter(j, b):
        pltpu.make_async_copy(bufs[b], acc.at[dst_v.at[j]], ssem[b]).wait()

    def steps(j, b):
        wait_gather(j, b)
        issue_scatter(j, b)
        wait_scatter(j - 1, (b + 1) % NBUF)
        issue_gather(j + 1, (b + 1) % NBUF)

    issue_gather(0, 0)
    wait_gather(0, 0)
    issue_scatter(0, 0)
    issue_gather(1, 1)
    steps(1, 1)

    def body(o, carry):
        j0 = o * NBUF
        for b in range(NBUF):
            steps(j0 + b, b)
        return carry

    lax.fori_loop(1, CPT // NBUF - 1, body, 0)

    for j in range(CPT - NBUF, CPT):
        b = j % NBUF
        if j + 1 < CPT:
            steps(j, b)
        else:
            wait_gather(j, b)
            issue_scatter(j, b)
            wait_scatter(j - 1, (b + 1) % NBUF)
    wait_scatter(CPT - 1, (CPT - 1) % NBUF)

    plsc.subcore_barrier()
    pltpu.sync_copy(acc.at[pl.ds(s * RPS, RPS)],
                    out_hbm.at[c, pl.ds(s * RPS, RPS)])


NB = 10
RB = N // NB

_blk = pl.BlockSpec((RB, D), lambda i: (i, 0))
_blkdeg = pl.BlockSpec((RB, DEGW), lambda i: (i, 0))
_blkw = pl.BlockSpec((D, D), lambda i: (0, 0))
_blkb = pl.BlockSpec((1, D), lambda i: (0, 0))
_tc_params = pltpu.CompilerParams(dimension_semantics=("parallel",))


def _dinv_of(dp0_ref, dp1_ref):
    return lax.rsqrt(1.0 + dp0_ref[:, 0:1] + dp1_ref[:, 0:1])


def _tc_first_body(dp0_ref, dp1_ref, x_ref, w_ref, o_ref):
    dinv = _dinv_of(dp0_ref, dp1_ref)
    h = jnp.dot(x_ref[...], w_ref[...], preferred_element_type=jnp.float32)
    o_ref[...] = h * dinv


def _tc_mid_body(dp0_ref, dp1_ref, g_ref, s0_ref, s1_ref, w_ref, b_ref, o_ref):
    dinv = _dinv_of(dp0_ref, dp1_ref)
    z = dinv * (g_ref[...] + s0_ref[...] + s1_ref[...]) + b_ref[...]
    z = jnp.maximum(z, 0.0)
    h = jnp.dot(z, w_ref[...], preferred_element_type=jnp.float32)
    o_ref[...] = h * dinv


def _tc_last_body(dp0_ref, dp1_ref, g_ref, s0_ref, s1_ref, w_ref, b_ref,
                  bh_ref, o_ref):
    dinv = _dinv_of(dp0_ref, dp1_ref)
    z = dinv * (g_ref[...] + s0_ref[...] + s1_ref[...]) + b_ref[...]
    h = jnp.dot(z, w_ref[...], preferred_element_type=jnp.float32)
    o_ref[...] = h + bh_ref[...]


_out_nd = jax.ShapeDtypeStruct((N, D), jnp.float32)

_tc_first = pl.pallas_call(
    _tc_first_body,
    grid=(NB,),
    in_specs=[_blkdeg, _blkdeg, _blk, _blkw],
    out_specs=_blk,
    out_shape=_out_nd,
    compiler_params=_tc_params,
)

_tc_mid = pl.pallas_call(
    _tc_mid_body,
    grid=(NB,),
    in_specs=[_blkdeg, _blkdeg, _blk, _blk, _blk, _blkw, _blkb],
    out_specs=_blk,
    out_shape=_out_nd,
    compiler_params=_tc_params,
)

_tc_last = pl.pallas_call(
    _tc_last_body,
    grid=(NB,),
    in_specs=[_blkdeg, _blkdeg, _blk, _blk, _blk, _blkw, _blkb, _blkb],
    out_specs=_blk,
    out_shape=_out_nd,
    compiler_params=_tc_params,
)


def kernel(x, edge_index, W1, b1, W2, b2, W3, b3, Wh, bh):
    pad = EP - E
    src2 = jnp.concatenate(
        [edge_index[0], jnp.zeros((pad,), jnp.int32)]).reshape(EP // B, B)
    dst2 = jnp.concatenate(
        [edge_index[1], jnp.full((pad,), N, jnp.int32)]).reshape(EP // B, B)
    zeros_d = jnp.zeros((NP, D), jnp.float32)
    zeros_w = jnp.zeros((NP, DEGW), jnp.float32)
    ones_w = jnp.ones((B, DEGW), jnp.float32)
    b1r = b1.reshape(1, D)
    b2r = b2.reshape(1, D)
    b3r = b3.reshape(1, D)
    bhr = bh.reshape(1, D)

    degp = _deg_sc(dst2, ones_w, zeros_w)
    dp0, dp1 = degp[0, :N], degp[1, :N]

    g1 = _tc_first(dp0, dp1, x, W1)
    s1 = _agg_sc(g1, src2, dst2, zeros_d)
    g2 = _tc_mid(dp0, dp1, g1, s1[0, :N], s1[1, :N], W2, b1r)
    s2 = _agg_sc(g2, src2, dst2, zeros_d)
    g3 = _tc_mid(dp0, dp1, g2, s2[0, :N], s2[1, :N], W3, b2r)
    s3 = _agg_sc(g3, src2, dst2, zeros_d)
    out = _tc_last(dp0, dp1, g3, s3[0, :N], s3[1, :N], Wh, b3r, bhr)
    return out


# speedup vs baseline: 6.3886x; 2.0536x over previous
import functools

import jax
import jax.numpy as jnp
from jax import lax
from jax.experimental import pallas as pl
from jax.experimental.pallas import tpu as pltpu
from jax.experimental.pallas import tpu_sc as plsc

N, D, E = 10000, 128, 320000
NC, NS = 2, 16
NW = NC * NS
B = 80
CPT = 125
EP = NW * B * CPT
NBUF = 3
NP = 10016
RPS = NP // NS
DEGW = 16

_mesh = plsc.VectorSubcoreMesh(
    core_axis_name="c", subcore_axis_name="s", num_cores=NC, num_subcores=NS)
_sc_params = pltpu.CompilerParams(use_tc_tiling_on_sc=False)


@functools.partial(
    pl.kernel,
    out_type=jax.ShapeDtypeStruct((NC, NP, DEGW), jnp.float32),
    mesh=_mesh,
    scratch_types=[
        pltpu.VMEM((B, DEGW), jnp.float32),
        pltpu.VMEM((CPT, B), jnp.int32),
        pltpu.VMEM_SHARED((NP, DEGW), jnp.float32),
    ],
    compiler_params=_sc_params,
)
def _deg_sc(dst_hbm, ones_hbm, zeros_hbm, out_hbm, ones_v, dst_v, acc):
    c = lax.axis_index("c")
    s = lax.axis_index("s")
    wid = c * NS + s
    pltpu.sync_copy(zeros_hbm.at[pl.ds(s * RPS, RPS)], acc.at[pl.ds(s * RPS, RPS)])
    pltpu.sync_copy(ones_hbm, ones_v)
    pltpu.sync_copy(dst_hbm.at[pl.ds(wid * CPT, CPT)], dst_v)
    plsc.subcore_barrier()

    def body(j, carry):
        pltpu.sync_copy(ones_v, acc.at[dst_v.at[j]], add=True)
        return carry

    lax.fori_loop(0, CPT, body, 0)
    plsc.subcore_barrier()
    pltpu.sync_copy(acc.at[pl.ds(s * RPS, RPS)],
                    out_hbm.at[c, pl.ds(s * RPS, RPS)])


@functools.partial(
    pl.kernel,
    out_type=jax.ShapeDtypeStruct((NC, NP, D), jnp.float32),
    mesh=_mesh,
    scratch_types=(
        [pltpu.VMEM((CPT, B), jnp.int32),
         pltpu.VMEM((CPT, B), jnp.int32)]
        + [pltpu.VMEM((B, D), jnp.float32) for _ in range(NBUF)]
        + [pltpu.VMEM_SHARED((NP, D), jnp.float32)]
        + [pltpu.SemaphoreType.DMA for _ in range(2 * NBUF)]
    ),
    compiler_params=_sc_params,
)
def _agg_sc(g_hbm, src_hbm, dst_hbm, zeros_hbm, out_hbm,
            src_v, dst_v, *rest):
    bufs = rest[:NBUF]
    acc = rest[NBUF]
    gsem = rest[NBUF + 1:NBUF + 1 + NBUF]
    ssem = rest[NBUF + 1 + NBUF:]
    c = lax.axis_index("c")
    s = lax.axis_index("s")
    wid = c * NS + s
    d0 = pltpu.async_copy(
        zeros_hbm.at[pl.ds(s * RPS, RPS)], acc.at[pl.ds(s * RPS, RPS)], gsem[0])
    d1 = pltpu.async_copy(src_hbm.at[pl.ds(wid * CPT, CPT)], src_v, gsem[1])
    d2 = pltpu.async_copy(dst_hbm.at[pl.ds(wid * CPT, CPT)], dst_v, ssem[0])
    d0.wait()
    d1.wait()
    d2.wait()
    plsc.subcore_barrier()

    def issue_gather(j, b):
        pltpu.async_copy(g_hbm.at[src_v.at[j]], bufs[b], gsem[b])

    def wait_gather(j, b):
        pltpu.make_async_copy(g_hbm.at[src_v.at[j]], bufs[b], gsem[b]).wait()

    def issue_scatter(j, b):
        pltpu.async_copy(bufs[b], acc.at[dst_v.at[j]], ssem[b], add=True)

    def wait_scatter(j, b):
        pltpu.make_async_copy(bufs[b], acc.at[dst_v.at[j]], ssem[b]).wait()

    def steps(j, b):
        wait_gather(j, b)
        issue_scatter(j, b)
        wait_scatter(j - 1, (b + 2) % NBUF)
        issue_gather(j + 2, (b + 2) % NBUF)

    issue_gather(0, 0)
    issue_gather(1, 1)
    wait_gather(0, 0)
    issue_scatter(0, 0)
    issue_gather(2, 2)
    for j in range(1, NBUF):
        steps(j, j)

    def body(o, carry):
        j0 = o * NBUF
        for b in range(NBUF):
            steps(j0 + b, b)
        return carry

    lax.fori_loop(1, CPT // NBUF - 1, body, 0)

    for j in range((CPT // NBUF - 1) * NBUF, CPT):
        b = j % NBUF
        if j + 2 < CPT:
            steps(j, b)
        else:
            wait_gather(j, b)
            issue_scatter(j, b)
            wait_scatter(j - 1, (b + 2) % NBUF)
    wait_scatter(CPT - 1, (CPT - 1) % NBUF)

    plsc.subcore_barrier()
    pltpu.sync_copy(acc.at[pl.ds(s * RPS, RPS)],
                    out_hbm.at[c, pl.ds(s * RPS, RPS)])


NB = 10
RB = N // NB

_blk = pl.BlockSpec((RB, D), lambda i: (i, 0))
_blkdeg = pl.BlockSpec((RB, DEGW), lambda i: (i, 0))
_blkw = pl.BlockSpec((D, D), lambda i: (0, 0))
_blkb = pl.BlockSpec((1, D), lambda i: (0, 0))
_tc_params = pltpu.CompilerParams(dimension_semantics=("parallel",))


def _dinv_of(dp0_ref, dp1_ref):
    return lax.rsqrt(1.0 + dp0_ref[:, 0:1] + dp1_ref[:, 0:1])


def _tc_first_body(dp0_ref, dp1_ref, x_ref, w_ref, o_ref):
    dinv = _dinv_of(dp0_ref, dp1_ref)
    h = jnp.dot(x_ref[...], w_ref[...], preferred_element_type=jnp.float32)
    o_ref[...] = h * dinv


def _tc_mid_body(dp0_ref, dp1_ref, g_ref, s0_ref, s1_ref, w_ref, b_ref, o_ref):
    dinv = _dinv_of(dp0_ref, dp1_ref)
    z = dinv * (g_ref[...] + s0_ref[...] + s1_ref[...]) + b_ref[...]
    z = jnp.maximum(z, 0.0)
    h = jnp.dot(z, w_ref[...], preferred_element_type=jnp.float32)
    o_ref[...] = h * dinv


def _tc_last_body(dp0_ref, dp1_ref, g_ref, s0_ref, s1_ref, w_ref, b_ref,
                  bh_ref, o_ref):
    dinv = _dinv_of(dp0_ref, dp1_ref)
    z = dinv * (g_ref[...] + s0_ref[...] + s1_ref[...]) + b_ref[...]
    h = jnp.dot(z, w_ref[...], preferred_element_type=jnp.float32)
    o_ref[...] = h + bh_ref[...]


_out_nd = jax.ShapeDtypeStruct((N, D), jnp.float32)

_tc_first = pl.pallas_call(
    _tc_first_body,
    grid=(NB,),
    in_specs=[_blkdeg, _blkdeg, _blk, _blkw],
    out_specs=_blk,
    out_shape=_out_nd,
    compiler_params=_tc_params,
)

_tc_mid = pl.pallas_call(
    _tc_mid_body,
    grid=(NB,),
    in_specs=[_blkdeg, _blkdeg, _blk, _blk, _blk, _blkw, _blkb],
    out_specs=_blk,
    out_shape=_out_nd,
    compiler_params=_tc_params,
)

_tc_last = pl.pallas_call(
    _tc_last_body,
    grid=(NB,),
    in_specs=[_blkdeg, _blkdeg, _blk, _blk, _blk, _blkw, _blkb, _blkb],
    out_specs=_blk,
    out_shape=_out_nd,
    compiler_params=_tc_params,
)


def kernel(x, edge_index, W1, b1, W2, b2, W3, b3, Wh, bh):
    pad = EP - E
    src2 = jnp.concatenate(
        [edge_index[0], jnp.zeros((pad,), jnp.int32)]).reshape(EP // B, B)
    dst2 = jnp.concatenate(
        [edge_index[1], jnp.full((pad,), N, jnp.int32)]).reshape(EP // B, B)
    zeros_d = jnp.zeros((NP, D), jnp.float32)
    zeros_w = jnp.zeros((NP, DEGW), jnp.float32)
    ones_w = jnp.ones((B, DEGW), jnp.float32)
    b1r = b1.reshape(1, D)
    b2r = b2.reshape(1, D)
    b3r = b3.reshape(1, D)
    bhr = bh.reshape(1, D)

    degp = _deg_sc(dst2, ones_w, zeros_w)
    dp0, dp1 = degp[0, :N], degp[1, :N]

    g1 = _tc_first(dp0, dp1, x, W1)
    s1 = _agg_sc(g1, src2, dst2, zeros_d)
    g2 = _tc_mid(dp0, dp1, g1, s1[0, :N], s1[1, :N], W2, b1r)
    s2 = _agg_sc(g2, src2, dst2, zeros_d)
    g3 = _tc_mid(dp0, dp1, g2, s2[0, :N], s2[1, :N], W3, b2r)
    s3 = _agg_sc(g3, src2, dst2, zeros_d)
    out = _tc_last(dp0, dp1, g3, s3[0, :N], s3[1, :N], Wh, b3r, bhr)
    return out


# speedup vs baseline: 7.7354x; 1.2108x over previous
import functools

import jax
import jax.numpy as jnp
from jax import lax
from jax.experimental import pallas as pl
from jax.experimental.pallas import tpu as pltpu
from jax.experimental.pallas import tpu_sc as plsc

N, D, E = 10000, 128, 320000
NC, NS = 2, 16
NW = NC * NS
B = 80
CPT = 125
EP = NW * B * CPT
NBUF = 5
NP = 10016
RPS = NP // NS
DEGW = 16

_mesh = plsc.VectorSubcoreMesh(
    core_axis_name="c", subcore_axis_name="s", num_cores=NC, num_subcores=NS)
_sc_params = pltpu.CompilerParams(use_tc_tiling_on_sc=False)


@functools.partial(
    pl.kernel,
    out_type=jax.ShapeDtypeStruct((NC, NP, DEGW), jnp.float32),
    mesh=_mesh,
    scratch_types=[
        pltpu.VMEM((B, DEGW), jnp.float32),
        pltpu.VMEM((CPT, B), jnp.int32),
        pltpu.VMEM_SHARED((NP, DEGW), jnp.float32),
    ],
    compiler_params=_sc_params,
)
def _deg_sc(dst_hbm, ones_hbm, zeros_hbm, out_hbm, ones_v, dst_v, acc):
    c = lax.axis_index("c")
    s = lax.axis_index("s")
    wid = c * NS + s
    pltpu.sync_copy(zeros_hbm.at[pl.ds(s * RPS, RPS)], acc.at[pl.ds(s * RPS, RPS)])
    pltpu.sync_copy(ones_hbm, ones_v)
    pltpu.sync_copy(dst_hbm.at[pl.ds(wid * CPT, CPT)], dst_v)
    plsc.subcore_barrier()

    def body(j, carry):
        pltpu.sync_copy(ones_v, acc.at[dst_v.at[j]], add=True)
        return carry

    lax.fori_loop(0, CPT, body, 0)
    plsc.subcore_barrier()
    pltpu.sync_copy(acc.at[pl.ds(s * RPS, RPS)],
                    out_hbm.at[c, pl.ds(s * RPS, RPS)])


@functools.partial(
    pl.kernel,
    out_type=jax.ShapeDtypeStruct((NC, NP, D), jnp.bfloat16),
    mesh=_mesh,
    scratch_types=(
        [pltpu.VMEM((CPT, B), jnp.int32),
         pltpu.VMEM((CPT, B), jnp.int32)]
        + [pltpu.VMEM((B, D), jnp.bfloat16) for _ in range(NBUF)]
        + [pltpu.VMEM_SHARED((NP, D), jnp.bfloat16)]
        + [pltpu.SemaphoreType.DMA for _ in range(2 * NBUF)]
    ),
    compiler_params=_sc_params,
)
def _agg_sc(g_hbm, src_hbm, dst_hbm, zeros_hbm, out_hbm,
            src_v, dst_v, *rest):
    bufs = rest[:NBUF]
    acc = rest[NBUF]
    gsem = rest[NBUF + 1:NBUF + 1 + NBUF]
    ssem = rest[NBUF + 1 + NBUF:]
    c = lax.axis_index("c")
    s = lax.axis_index("s")
    wid = c * NS + s
    d0 = pltpu.async_copy(
        zeros_hbm.at[pl.ds(s * RPS, RPS)], acc.at[pl.ds(s * RPS, RPS)], gsem[0])
    d1 = pltpu.async_copy(src_hbm.at[pl.ds(wid * CPT, CPT)], src_v, gsem[1])
    d2 = pltpu.async_copy(dst_hbm.at[pl.ds(wid * CPT, CPT)], dst_v, ssem[0])
    d0.wait()
    d1.wait()
    d2.wait()
    plsc.subcore_barrier()

    def issue_gather(j, b):
        pltpu.async_copy(g_hbm.at[src_v.at[j]], bufs[b], gsem[b])

    def wait_gather(j, b):
        pltpu.make_async_copy(g_hbm.at[src_v.at[j]], bufs[b], gsem[b]).wait()

    def issue_scatter(j, b):
        pltpu.async_copy(bufs[b], acc.at[dst_v.at[j]], ssem[b], add=True)

    def wait_scatter(j, b):
        pltpu.make_async_copy(bufs[b], acc.at[dst_v.at[j]], ssem[b]).wait()

    def steps(j, b):
        wait_gather(j, b)
        issue_scatter(j, b)
        wait_scatter(j - 1, (b + NBUF - 1) % NBUF)
        issue_gather(j + NBUF - 1, (b + NBUF - 1) % NBUF)

    for j in range(NBUF - 1):
        issue_gather(j, j)
    wait_gather(0, 0)
    issue_scatter(0, 0)
    issue_gather(NBUF - 1, NBUF - 1)
    for j in range(1, NBUF):
        steps(j, j)

    def body(o, carry):
        j0 = o * NBUF
        for b in range(NBUF):
            steps(j0 + b, b)
        return carry

    lax.fori_loop(1, CPT // NBUF - 1, body, 0)

    for j in range((CPT // NBUF - 1) * NBUF, CPT):
        b = j % NBUF
        if j + NBUF - 1 < CPT:
            steps(j, b)
        else:
            wait_gather(j, b)
            issue_scatter(j, b)
            wait_scatter(j - 1, (b + NBUF - 1) % NBUF)
    wait_scatter(CPT - 1, (CPT - 1) % NBUF)

    plsc.subcore_barrier()
    pltpu.sync_copy(acc.at[pl.ds(s * RPS, RPS)],
                    out_hbm.at[c, pl.ds(s * RPS, RPS)])


NB = 5
RB = N // NB

_blk = pl.BlockSpec((RB, D), lambda i: (i, 0))
_blks = pl.BlockSpec((NC, RB, D), lambda i: (0, i, 0))
_blkdeg = pl.BlockSpec((RB, DEGW), lambda i: (i, 0))
_blkw = pl.BlockSpec((D, D), lambda i: (0, 0))
_blkb = pl.BlockSpec((1, D), lambda i: (0, 0))
_tc_params = pltpu.CompilerParams(dimension_semantics=("parallel",))


def _dinv_of(dp0_ref, dp1_ref):
    return lax.rsqrt(1.0 + dp0_ref[:, 0:1] + dp1_ref[:, 0:1])


def _tc_first_body(dp0_ref, dp1_ref, x_ref, w_ref, o_ref, ob_ref):
    dinv = _dinv_of(dp0_ref, dp1_ref)
    h = jnp.dot(x_ref[...], w_ref[...], preferred_element_type=jnp.float32)
    g = h * dinv
    o_ref[...] = g
    ob_ref[...] = g.astype(jnp.bfloat16)


def _tc_mid_body(dp0_ref, dp1_ref, g_ref, s_ref, w_ref, b_ref, o_ref, ob_ref):
    dinv = _dinv_of(dp0_ref, dp1_ref)
    sagg = s_ref[0].astype(jnp.float32) + s_ref[1].astype(jnp.float32)
    z = dinv * (g_ref[...] + sagg) + b_ref[...]
    z = jnp.maximum(z, 0.0)
    h = jnp.dot(z, w_ref[...], preferred_element_type=jnp.float32)
    g = h * dinv
    o_ref[...] = g
    ob_ref[...] = g.astype(jnp.bfloat16)


def _tc_last_body(dp0_ref, dp1_ref, g_ref, s_ref, w_ref, b_ref, bh_ref, o_ref):
    dinv = _dinv_of(dp0_ref, dp1_ref)
    sagg = s_ref[0].astype(jnp.float32) + s_ref[1].astype(jnp.float32)
    z = dinv * (g_ref[...] + sagg) + b_ref[...]
    h = jnp.dot(z, w_ref[...], preferred_element_type=jnp.float32)
    o_ref[...] = h + bh_ref[...]


_out_f32 = jax.ShapeDtypeStruct((N, D), jnp.float32)
_out_bf16 = jax.ShapeDtypeStruct((N, D), jnp.bfloat16)

_tc_first = pl.pallas_call(
    _tc_first_body,
    grid=(NB,),
    in_specs=[_blkdeg, _blkdeg, _blk, _blkw],
    out_specs=[_blk, _blk],
    out_shape=[_out_f32, _out_bf16],
    compiler_params=_tc_params,
)

_tc_mid = pl.pallas_call(
    _tc_mid_body,
    grid=(NB,),
    in_specs=[_blkdeg, _blkdeg, _blk, _blks, _blkw, _blkb],
    out_specs=[_blk, _blk],
    out_shape=[_out_f32, _out_bf16],
    compiler_params=_tc_params,
)

_tc_last = pl.pallas_call(
    _tc_last_body,
    grid=(NB,),
    in_specs=[_blkdeg, _blkdeg, _blk, _blks, _blkw, _blkb, _blkb],
    out_specs=_blk,
    out_shape=_out_f32,
    compiler_params=_tc_params,
)


def kernel(x, edge_index, W1, b1, W2, b2, W3, b3, Wh, bh):
    src2 = edge_index[0].reshape(EP // B, B)
    dst2 = edge_index[1].reshape(EP // B, B)
    zeros_d = jnp.zeros((NP, D), jnp.bfloat16)
    zeros_w = jnp.zeros((NP, DEGW), jnp.float32)
    ones_w = jnp.ones((B, DEGW), jnp.float32)
    b1r = b1.reshape(1, D)
    b2r = b2.reshape(1, D)
    b3r = b3.reshape(1, D)
    bhr = bh.reshape(1, D)

    degp = _deg_sc(dst2, ones_w, zeros_w)
    dp0, dp1 = degp[0, :N], degp[1, :N]

    g1, g1b = _tc_first(dp0, dp1, x, W1)
    s1 = _agg_sc(g1b, src2, dst2, zeros_d)
    g2, g2b = _tc_mid(dp0, dp1, g1, s1[:, :N], W2, b1r)
    s2 = _agg_sc(g2b, src2, dst2, zeros_d)
    g3, g3b = _tc_mid(dp0, dp1, g2, s2[:, :N], W3, b2r)
    s3 = _agg_sc(g3b, src2, dst2, zeros_d)
    out = _tc_last(dp0, dp1, g3, s3[:, :N], Wh, b3r, bhr)
    return out


# speedup vs baseline: 7.7388x; 1.0004x over previous
import functools

import jax
import jax.numpy as jnp
from jax import lax
from jax.experimental import pallas as pl
from jax.experimental.pallas import tpu as pltpu
from jax.experimental.pallas import tpu_sc as plsc

N, D, E = 10000, 128, 320000
NC, NS = 2, 16
NW = NC * NS
B = 80
CPT = 125
EP = NW * B * CPT
NBUF = 5
NP = 10016
RPS = NP // NS
DEGW = 16

_mesh = plsc.VectorSubcoreMesh(
    core_axis_name="c", subcore_axis_name="s", num_cores=NC, num_subcores=NS)
_sc_params = pltpu.CompilerParams(use_tc_tiling_on_sc=False)


@functools.partial(
    pl.kernel,
    out_type=jax.ShapeDtypeStruct((NC, NP, DEGW), jnp.float32),
    mesh=_mesh,
    scratch_types=[
        pltpu.VMEM((B, DEGW), jnp.float32),
        pltpu.VMEM((CPT, B), jnp.int32),
        pltpu.VMEM_SHARED((NP, DEGW), jnp.float32),
    ],
    compiler_params=_sc_params,
)
def _deg_sc(dst_hbm, ones_hbm, zeros_hbm, out_hbm, ones_v, dst_v, acc):
    c = lax.axis_index("c")
    s = lax.axis_index("s")
    wid = c * NS + s
    pltpu.sync_copy(zeros_hbm.at[pl.ds(s * RPS, RPS)], acc.at[pl.ds(s * RPS, RPS)])
    pltpu.sync_copy(ones_hbm, ones_v)
    pltpu.sync_copy(dst_hbm.at[pl.ds(wid * CPT, CPT)], dst_v)
    plsc.subcore_barrier()

    def body(j, carry):
        pltpu.sync_copy(ones_v, acc.at[dst_v.at[j]], add=True)
        return carry

    lax.fori_loop(0, CPT, body, 0)
    plsc.subcore_barrier()
    pltpu.sync_copy(acc.at[pl.ds(s * RPS, RPS)],
                    out_hbm.at[c, pl.ds(s * RPS, RPS)])


@functools.partial(
    pl.kernel,
    out_type=jax.ShapeDtypeStruct((NC, NP, D), jnp.bfloat16),
    mesh=_mesh,
    scratch_types=(
        [pltpu.VMEM((CPT, B), jnp.int32),
         pltpu.VMEM((CPT, B), jnp.int32)]
        + [pltpu.VMEM((B, D), jnp.bfloat16) for _ in range(NBUF)]
        + [pltpu.VMEM_SHARED((NP, D), jnp.bfloat16)]
        + [pltpu.SemaphoreType.DMA for _ in range(2 * NBUF)]
    ),
    compiler_params=_sc_params,
)
def _agg_sc(g_hbm, src_hbm, dst_hbm, zeros_hbm, out_hbm,
            src_v, dst_v, *rest):
    bufs = rest[:NBUF]
    acc = rest[NBUF]
    gsem = rest[NBUF + 1:NBUF + 1 + NBUF]
    ssem = rest[NBUF + 1 + NBUF:]
    c = lax.axis_index("c")
    s = lax.axis_index("s")
    wid = c * NS + s
    d0 = pltpu.async_copy(
        zeros_hbm.at[pl.ds(s * RPS, RPS)], acc.at[pl.ds(s * RPS, RPS)], gsem[0])
    d1 = pltpu.async_copy(src_hbm.at[pl.ds(wid * CPT, CPT)], src_v, gsem[1])
    d2 = pltpu.async_copy(dst_hbm.at[pl.ds(wid * CPT, CPT)], dst_v, ssem[0])
    d0.wait()
    d1.wait()
    d2.wait()
    plsc.subcore_barrier()

    def issue_gather(j, b):
        pltpu.async_copy(g_hbm.at[src_v.at[j]], bufs[b], gsem[b])

    def wait_gather(j, b):
        pltpu.make_async_copy(g_hbm.at[src_v.at[j]], bufs[b], gsem[b]).wait()

    def issue_scatter(j, b):
        pltpu.async_copy(bufs[b], acc.at[dst_v.at[j]], ssem[b], add=True)

    def wait_scatter(j, b):
        pltpu.make_async_copy(bufs[b], acc.at[dst_v.at[j]], ssem[b]).wait()

    A = NBUF - 1
    for j in range(A):
        issue_gather(j, j)

    def body(o, carry):
        for b in range(NBUF):
            j = o * NBUF + b

            @pl.when(j < CPT)
            def _():
                wait_gather(j, b)
                issue_scatter(j, b)

            @pl.when(jnp.logical_and(j >= 1, j <= CPT))
            def _():
                wait_scatter(j - 1, (b + A) % NBUF)

            @pl.when(j + A < CPT)
            def _():
                issue_gather(j + A, (b + A) % NBUF)
        return carry

    lax.fori_loop(0, (CPT + NBUF) // NBUF, body, 0)

    plsc.subcore_barrier()
    pltpu.sync_copy(acc.at[pl.ds(s * RPS, RPS)],
                    out_hbm.at[c, pl.ds(s * RPS, RPS)])


NB = 5
RB = N // NB

_blk = pl.BlockSpec((RB, D), lambda i: (i, 0))
_blks = pl.BlockSpec((NC, RB, D), lambda i: (0, i, 0))
_blkdeg = pl.BlockSpec((RB, DEGW), lambda i: (i, 0))
_blkw = pl.BlockSpec((D, D), lambda i: (0, 0))
_blkb = pl.BlockSpec((1, D), lambda i: (0, 0))
_tc_params = pltpu.CompilerParams(dimension_semantics=("parallel",))


def _dinv_of(dp0_ref, dp1_ref):
    return lax.rsqrt(1.0 + dp0_ref[:, 0:1] + dp1_ref[:, 0:1])


def _tc_first_body(dp0_ref, dp1_ref, x_ref, w_ref, o_ref, ob_ref):
    dinv = _dinv_of(dp0_ref, dp1_ref)
    h = jnp.dot(x_ref[...], w_ref[...], preferred_element_type=jnp.float32)
    g = h * dinv
    o_ref[...] = g
    ob_ref[...] = g.astype(jnp.bfloat16)


def _tc_mid_body(dp0_ref, dp1_ref, g_ref, s_ref, w_ref, b_ref, o_ref, ob_ref):
    dinv = _dinv_of(dp0_ref, dp1_ref)
    sagg = s_ref[0].astype(jnp.float32) + s_ref[1].astype(jnp.float32)
    z = dinv * (g_ref[...] + sagg) + b_ref[...]
    z = jnp.maximum(z, 0.0)
    h = jnp.dot(z, w_ref[...], preferred_element_type=jnp.float32)
    g = h * dinv
    o_ref[...] = g
    ob_ref[...] = g.astype(jnp.bfloat16)


def _tc_last_body(dp0_ref, dp1_ref, g_ref, s_ref, w_ref, b_ref, bh_ref, o_ref):
    dinv = _dinv_of(dp0_ref, dp1_ref)
    sagg = s_ref[0].astype(jnp.float32) + s_ref[1].astype(jnp.float32)
    z = dinv * (g_ref[...] + sagg) + b_ref[...]
    h = jnp.dot(z, w_ref[...], preferred_element_type=jnp.float32)
    o_ref[...] = h + bh_ref[...]


_out_f32 = jax.ShapeDtypeStruct((N, D), jnp.float32)
_out_bf16 = jax.ShapeDtypeStruct((N, D), jnp.bfloat16)

_tc_first = pl.pallas_call(
    _tc_first_body,
    grid=(NB,),
    in_specs=[_blkdeg, _blkdeg, _blk, _blkw],
    out_specs=[_blk, _blk],
    out_shape=[_out_f32, _out_bf16],
    compiler_params=_tc_params,
)

_tc_mid = pl.pallas_call(
    _tc_mid_body,
    grid=(NB,),
    in_specs=[_blkdeg, _blkdeg, _blk, _blks, _blkw, _blkb],
    out_specs=[_blk, _blk],
    out_shape=[_out_f32, _out_bf16],
    compiler_params=_tc_params,
)

_tc_last = pl.pallas_call(
    _tc_last_body,
    grid=(NB,),
    in_specs=[_blkdeg, _blkdeg, _blk, _blks, _blkw, _blkb, _blkb],
    out_specs=_blk,
    out_shape=_out_f32,
    compiler_params=_tc_params,
)


def kernel(x, edge_index, W1, b1, W2, b2, W3, b3, Wh, bh):
    src2 = edge_index[0].reshape(EP // B, B)
    dst2 = edge_index[1].reshape(EP // B, B)
    zeros_d = jnp.zeros((NP, D), jnp.bfloat16)
    zeros_w = jnp.zeros((NP, DEGW), jnp.float32)
    ones_w = jnp.ones((B, DEGW), jnp.float32)
    b1r = b1.reshape(1, D)
    b2r = b2.reshape(1, D)
    b3r = b3.reshape(1, D)
    bhr = bh.reshape(1, D)

    degp = _deg_sc(dst2, ones_w, zeros_w)
    dp0, dp1 = degp[0, :N], degp[1, :N]

    g1, g1b = _tc_first(dp0, dp1, x, W1)
    s1 = _agg_sc(g1b, src2, dst2, zeros_d)
    g2, g2b = _tc_mid(dp0, dp1, g1, s1[:, :N], W2, b1r)
    s2 = _agg_sc(g2b, src2, dst2, zeros_d)
    g3, g3b = _tc_mid(dp0, dp1, g2, s2[:, :N], W3, b2r)
    s3 = _agg_sc(g3b, src2, dst2, zeros_d)
    out = _tc_last(dp0, dp1, g3, s3[:, :N], Wh, b3r, bhr)
    return out


# speedup vs baseline: 7.9152x; 1.0228x over previous
import functools

import jax
import jax.numpy as jnp
from jax import lax
from jax.experimental import pallas as pl
from jax.experimental.pallas import tpu as pltpu
from jax.experimental.pallas import tpu_sc as plsc

N, D, E = 10000, 128, 320000
NC, NS = 2, 16
NW = NC * NS
B = 80
CPT = 125
EP = NW * B * CPT
NBUF = 5
NP = 10016
RPS = NP // NS
DEGW = 16

_mesh = plsc.VectorSubcoreMesh(
    core_axis_name="c", subcore_axis_name="s", num_cores=NC, num_subcores=NS)
_sc_params = pltpu.CompilerParams(use_tc_tiling_on_sc=False)


@functools.partial(
    pl.kernel,
    out_type=jax.ShapeDtypeStruct((NC, NP, DEGW), jnp.float32),
    mesh=_mesh,
    scratch_types=[
        pltpu.VMEM((B, DEGW), jnp.float32),
        pltpu.VMEM((CPT, B), jnp.int32),
        pltpu.VMEM_SHARED((NP, DEGW), jnp.float32),
    ],
    compiler_params=_sc_params,
)
def _deg_sc(dst_hbm, ones_hbm, zeros_hbm, out_hbm, ones_v, dst_v, acc):
    c = lax.axis_index("c")
    s = lax.axis_index("s")
    wid = c * NS + s
    pltpu.sync_copy(zeros_hbm.at[pl.ds(s * RPS, RPS)], acc.at[pl.ds(s * RPS, RPS)])
    pltpu.sync_copy(ones_hbm, ones_v)
    pltpu.sync_copy(dst_hbm.at[pl.ds(wid * CPT, CPT)], dst_v)
    plsc.subcore_barrier()

    def body(j, carry):
        pltpu.sync_copy(ones_v, acc.at[dst_v.at[j]], add=True)
        return carry

    lax.fori_loop(0, CPT, body, 0)
    plsc.subcore_barrier()
    pltpu.sync_copy(acc.at[pl.ds(s * RPS, RPS)],
                    out_hbm.at[c, pl.ds(s * RPS, RPS)])


@functools.partial(
    pl.kernel,
    out_type=jax.ShapeDtypeStruct((NC, NP, D), jnp.bfloat16),
    mesh=_mesh,
    scratch_types=(
        [pltpu.VMEM((CPT, B), jnp.int32),
         pltpu.VMEM((CPT, B), jnp.int32)]
        + [pltpu.VMEM((B, D), jnp.bfloat16) for _ in range(NBUF)]
        + [pltpu.VMEM_SHARED((NP, D), jnp.bfloat16)]
        + [pltpu.SemaphoreType.DMA for _ in range(2 * NBUF)]
    ),
    compiler_params=_sc_params,
)
def _agg_sc(g_hbm, src_hbm, dst_hbm, zeros_hbm, out_hbm,
            src_v, dst_v, *rest):
    bufs = rest[:NBUF]
    acc = rest[NBUF]
    gsem = rest[NBUF + 1:NBUF + 1 + NBUF]
    ssem = rest[NBUF + 1 + NBUF:]
    c = lax.axis_index("c")
    s = lax.axis_index("s")
    wid = c * NS + s
    d0 = pltpu.async_copy(
        zeros_hbm.at[pl.ds(s * RPS, RPS)], acc.at[pl.ds(s * RPS, RPS)], gsem[0])
    d1 = pltpu.async_copy(src_hbm.at[pl.ds(wid * CPT, CPT)], src_v, gsem[1])
    d2 = pltpu.async_copy(dst_hbm.at[pl.ds(wid * CPT, CPT)], dst_v, ssem[0])
    d0.wait()
    d1.wait()
    d2.wait()
    plsc.subcore_barrier()

    def issue_gather(j, b):
        pltpu.async_copy(g_hbm.at[src_v.at[j]], bufs[b], gsem[b])

    def wait_gather(j, b):
        pltpu.make_async_copy(g_hbm.at[src_v.at[j]], bufs[b], gsem[b]).wait()

    def issue_scatter(j, b):
        pltpu.async_copy(bufs[b], acc.at[dst_v.at[j]], ssem[b], add=True)

    def wait_scatter(j, b):
        pltpu.make_async_copy(bufs[b], acc.at[dst_v.at[j]], ssem[b]).wait()

    A = NBUF - 1
    for j in range(A):
        issue_gather(j, j)

    def body(o, carry):
        for b in range(NBUF):
            j = o * NBUF + b

            @pl.when(j < CPT)
            def _():
                wait_gather(j, b)
                issue_scatter(j, b)

            @pl.when(jnp.logical_and(j >= 1, j <= CPT))
            def _():
                wait_scatter(j - 1, (b + A) % NBUF)

            @pl.when(j + A < CPT)
            def _():
                issue_gather(j + A, (b + A) % NBUF)
        return carry

    lax.fori_loop(0, (CPT + NBUF) // NBUF, body, 0)

    plsc.subcore_barrier()
    pltpu.sync_copy(acc.at[pl.ds(s * RPS, RPS)],
                    out_hbm.at[c, pl.ds(s * RPS, RPS)])


NB = 5
RB = N // NB

_blk = pl.BlockSpec((RB, D), lambda i: (i, 0))
_blks = pl.BlockSpec((NC, RB, D), lambda i: (0, i, 0))
_blkdeg0 = pl.BlockSpec((1, RB, DEGW), lambda i: (0, i, 0))
_blkdeg1 = pl.BlockSpec((1, RB, DEGW), lambda i: (1, i, 0))
_blkw = pl.BlockSpec((D, D), lambda i: (0, 0))
_blkb = pl.BlockSpec((1, D), lambda i: (0, 0))
_tc_params = pltpu.CompilerParams(dimension_semantics=("parallel",))


def _dinv_of(dp0_ref, dp1_ref):
    return lax.rsqrt(1.0 + dp0_ref[0, :, 0:1] + dp1_ref[0, :, 0:1])


def _tc_first_body(dp0_ref, dp1_ref, x_ref, w_ref, o_ref, ob_ref):
    dinv = _dinv_of(dp0_ref, dp1_ref)
    h = jnp.dot(x_ref[...], w_ref[...], preferred_element_type=jnp.float32)
    g = h * dinv
    o_ref[...] = g
    ob_ref[...] = g.astype(jnp.bfloat16)


def _tc_mid_body(dp0_ref, dp1_ref, g_ref, s_ref, w_ref, b_ref, o_ref, ob_ref):
    dinv = _dinv_of(dp0_ref, dp1_ref)
    sagg = s_ref[0].astype(jnp.float32) + s_ref[1].astype(jnp.float32)
    z = dinv * (g_ref[...] + sagg) + b_ref[...]
    z = jnp.maximum(z, 0.0)
    h = jnp.dot(z, w_ref[...], preferred_element_type=jnp.float32)
    g = h * dinv
    o_ref[...] = g
    ob_ref[...] = g.astype(jnp.bfloat16)


def _tc_last_body(dp0_ref, dp1_ref, g_ref, s_ref, w_ref, b_ref, bh_ref, o_ref):
    dinv = _dinv_of(dp0_ref, dp1_ref)
    sagg = s_ref[0].astype(jnp.float32) + s_ref[1].astype(jnp.float32)
    z = dinv * (g_ref[...] + sagg) + b_ref[...]
    h = jnp.dot(z, w_ref[...], preferred_element_type=jnp.float32)
    o_ref[...] = h + bh_ref[...]


_out_f32 = jax.ShapeDtypeStruct((N, D), jnp.float32)
_out_bf16 = jax.ShapeDtypeStruct((N, D), jnp.bfloat16)

_tc_first = pl.pallas_call(
    _tc_first_body,
    grid=(NB,),
    in_specs=[_blkdeg0, _blkdeg1, _blk, _blkw],
    out_specs=[_blk, _blk],
    out_shape=[_out_f32, _out_bf16],
    compiler_params=_tc_params,
)

_tc_mid = pl.pallas_call(
    _tc_mid_body,
    grid=(NB,),
    in_specs=[_blkdeg0, _blkdeg1, _blk, _blks, _blkw, _blkb],
    out_specs=[_blk, _blk],
    out_shape=[_out_f32, _out_bf16],
    compiler_params=_tc_params,
)

_tc_last = pl.pallas_call(
    _tc_last_body,
    grid=(NB,),
    in_specs=[_blkdeg0, _blkdeg1, _blk, _blks, _blkw, _blkb, _blkb],
    out_specs=_blk,
    out_shape=_out_f32,
    compiler_params=_tc_params,
)


def kernel(x, edge_index, W1, b1, W2, b2, W3, b3, Wh, bh):
    src2 = edge_index[0].reshape(EP // B, B)
    dst2 = edge_index[1].reshape(EP // B, B)
    zeros_d = jnp.zeros((NP, D), jnp.bfloat16)
    zeros_w = jnp.zeros((NP, DEGW), jnp.float32)
    ones_w = jnp.ones((B, DEGW), jnp.float32)
    b1r = b1.reshape(1, D)
    b2r = b2.reshape(1, D)
    b3r = b3.reshape(1, D)
    bhr = bh.reshape(1, D)

    degp = _deg_sc(dst2, ones_w, zeros_w)

    g1, g1b = _tc_first(degp, degp, x, W1)
    s1 = _agg_sc(g1b, src2, dst2, zeros_d)
    g2, g2b = _tc_mid(degp, degp, g1, s1, W2, b1r)
    s2 = _agg_sc(g2b, src2, dst2, zeros_d)
    g3, g3b = _tc_mid(degp, degp, g2, s2, W3, b2r)
    s3 = _agg_sc(g3b, src2, dst2, zeros_d)
    out = _tc_last(degp, degp, g3, s3, Wh, b3r, bhr)
    return out
